# baseline (speedup 1.0000x reference)
import functools

import jax
import jax.numpy as jnp
from jax import lax
from jax.experimental import pallas as pl
from jax.experimental.pallas import tpu as pltpu
from jax.experimental.pallas import tpu_sc as plsc

_NUM_POINTS = 1000000
_PT_DIM = 64
_N_REP = 16384
_NC, _NS = 2, 16
_NW = _NC * _NS
_ROWS_PER_W = _PT_DIM // _NW
_CHUNK = 128
_NCHUNK = _N_REP // _CHUNK
_GROUP = 8

_COPY_ROWS = _NUM_POINTS * _PT_DIM // 128
_BLK = 5000


def _copy_body(src_ref, dst_ref):
    dst_ref[...] = src_ref[...]


@jax.jit
def _tc_copy(x):
    return pl.pallas_call(
        _copy_body,
        out_shape=jax.ShapeDtypeStruct((_COPY_ROWS, 128), jnp.float32),
        grid=(_COPY_ROWS // _BLK,),
        in_specs=[pl.BlockSpec((_BLK, 128), lambda i: (i, 0))],
        out_specs=pl.BlockSpec((_BLK, 128), lambda i: (i, 0)),
    )(x)


@functools.cache
def _make_scatter():
    mesh = plsc.VectorSubcoreMesh(
        core_axis_name="c", subcore_axis_name="s", num_cores=_NC, num_subcores=_NS
    )

    @functools.partial(
        pl.kernel,
        mesh=mesh,
        compiler_params=pltpu.CompilerParams(use_tc_tiling_on_sc=False),
        scratch_types=[
            pltpu.VMEM((_NCHUNK, _CHUNK), jnp.int32),
            pltpu.VMEM((_ROWS_PER_W, _N_REP), jnp.float32),
            pltpu.SemaphoreType.DMA,
        ],
    )
    def _scatter_cols(outT_ref, rvT_hbm, idx_hbm, idx_v, val_v, sc_sem):
        w = lax.axis_index("s") * _NC + lax.axis_index("c")
        base = w * _ROWS_PER_W
        pltpu.sync_copy(idx_hbm, idx_v)
        pltpu.sync_copy(rvT_hbm.at[pl.ds(base, _ROWS_PER_W)], val_v)

        for j in range(_ROWS_PER_W):
            r = base + j

            @pl.loop(0, _NCHUNK // _GROUP)
            def _(g):
                for b in range(_GROUP):
                    q = g * _GROUP + b
                    pltpu.async_copy(
                        val_v.at[j, pl.ds(q * _CHUNK, _CHUNK)],
                        outT_ref.at[r].at[idx_v.at[q]],
                        sc_sem,
                    )

        for j in range(_ROWS_PER_W):
            pltpu.make_async_copy(
                rvT_hbm.at[0, pl.ds(0, _N_REP)], val_v.at[j], sc_sem
            ).wait()

    return _scatter_cols


def kernel(pts, rand_vals, idx):
    ptsT = pts.T
    rvT = rand_vals.T
    idx2 = idx.astype(jnp.int32).reshape(_NCHUNK, _CHUNK)
    copied = _tc_copy(ptsT.reshape(_COPY_ROWS, 128))
    outT_ref = jax.new_ref(copied.reshape(_PT_DIM, _NUM_POINTS))
    _make_scatter()(outT_ref, rvT, idx2)
    return jax.freeze(outT_ref).T


# speedup vs baseline: 1.1452x; 1.1452x over previous
import functools

import jax
import jax.numpy as jnp
from jax import lax
from jax.experimental import pallas as pl
from jax.experimental.pallas import tpu as pltpu
from jax.experimental.pallas import tpu_sc as plsc

_NUM_POINTS = 1000000
_PT_DIM = 64
_N_REP = 16384
_NC, _NS = 2, 16
_NW = _NC * _NS
_ROWS_PER_W = _PT_DIM // _NW
_CHUNK = 128
_NCHUNK = _N_REP // _CHUNK
_GROUP = 8

_BLK = 4096


def _copy_body(src_ref, dst_ref):
    dst_ref[...] = src_ref[...]


def _tc_copy(x):
    return pl.pallas_call(
        _copy_body,
        out_shape=jax.ShapeDtypeStruct((_PT_DIM, _NUM_POINTS), jnp.float32),
        grid=(pl.cdiv(_NUM_POINTS, _BLK),),
        in_specs=[pl.BlockSpec((_PT_DIM, _BLK), lambda i: (0, i))],
        out_specs=pl.BlockSpec((_PT_DIM, _BLK), lambda i: (0, i)),
    )(x)


@functools.cache
def _make_scatter():
    mesh = plsc.VectorSubcoreMesh(
        core_axis_name="c", subcore_axis_name="s", num_cores=_NC, num_subcores=_NS
    )

    @functools.partial(
        pl.kernel,
        mesh=mesh,
        compiler_params=pltpu.CompilerParams(use_tc_tiling_on_sc=False),
        scratch_types=[
            pltpu.VMEM((_NCHUNK, _CHUNK), jnp.int32),
            pltpu.VMEM((_ROWS_PER_W, _N_REP), jnp.float32),
            pltpu.SemaphoreType.DMA,
        ],
    )
    def _scatter_cols(outT_ref, rvT_hbm, idx_hbm, idx_v, val_v, sc_sem):
        w = lax.axis_index("s") * _NC + lax.axis_index("c")
        base = w * _ROWS_PER_W
        pltpu.sync_copy(idx_hbm, idx_v)
        pltpu.sync_copy(rvT_hbm.at[pl.ds(base, _ROWS_PER_W)], val_v)

        for j in range(_ROWS_PER_W):
            r = base + j

            @pl.loop(0, _NCHUNK // _GROUP)
            def _(g):
                for b in range(_GROUP):
                    q = g * _GROUP + b
                    pltpu.async_copy(
                        val_v.at[j, pl.ds(q * _CHUNK, _CHUNK)],
                        outT_ref.at[r].at[idx_v.at[q]],
                        sc_sem,
                    )

        for j in range(_ROWS_PER_W):
            pltpu.make_async_copy(
                rvT_hbm.at[0, pl.ds(0, _N_REP)], val_v.at[j], sc_sem
            ).wait()

    return _scatter_cols


def kernel(pts, rand_vals, idx):
    ptsT = pts.T
    rvT = rand_vals.T
    idx2 = idx.astype(jnp.int32).reshape(_NCHUNK, _CHUNK)
    copied = _tc_copy(ptsT)
    outT_ref = jax.new_ref(copied)
    _make_scatter()(outT_ref, rvT, idx2)
    return jax.freeze(outT_ref).T


# speedup vs baseline: 12.2165x; 10.6678x over previous
import functools

import jax
import jax.numpy as jnp
from jax import lax
from jax.experimental import pallas as pl
from jax.experimental.pallas import tpu as pltpu
from jax.experimental.pallas import tpu_sc as plsc

_NUM_POINTS = 1000000
_PT_DIM = 64
_N_REP = 16384
_NC, _NS = 2, 16
_NW = _NC * _NS
_ROWS_PER_W = _N_REP // _NW
_CHUNK = 128
_CHUNKS_PER_W = _ROWS_PER_W // _CHUNK


@functools.cache
def _make_scatter():
    mesh = plsc.VectorSubcoreMesh(
        core_axis_name="c", subcore_axis_name="s", num_cores=_NC, num_subcores=_NS
    )

    @functools.partial(
        pl.kernel,
        mesh=mesh,
        compiler_params=pltpu.CompilerParams(use_tc_tiling_on_sc=False),
        scratch_types=[
            pltpu.VMEM((_CHUNKS_PER_W, _CHUNK), jnp.int32),
            pltpu.VMEM((_ROWS_PER_W, _PT_DIM), jnp.float32),
            pltpu.SemaphoreType.DMA,
        ],
    )
    def _scatter_rows(pts_ref, rv_hbm, idx_hbm, idx_v, rows_v, sem):
        w = lax.axis_index("s") * _NC + lax.axis_index("c")
        pltpu.sync_copy(idx_hbm.at[pl.ds(w * _CHUNKS_PER_W, _CHUNKS_PER_W)], idx_v)
        pltpu.sync_copy(rv_hbm.at[pl.ds(w * _ROWS_PER_W, _ROWS_PER_W)], rows_v)
        copies = [
            pltpu.async_copy(
                rows_v.at[pl.ds(j * _CHUNK, _CHUNK)],
                pts_ref.at[idx_v.at[j]],
                sem,
            )
            for j in range(_CHUNKS_PER_W)
        ]
        for c in copies:
            c.wait()

    return _scatter_rows


def kernel(pts, rand_vals, idx):
    idx2 = idx.astype(jnp.int32).reshape(_NW * _CHUNKS_PER_W, _CHUNK)
    pts_ref = jax.new_ref(pts)
    _make_scatter()(pts_ref, rand_vals, idx2)
    return jax.freeze(pts_ref)
